# baseline (speedup 1.0000x reference)
import functools
import math

import jax
import jax.numpy as jnp
from jax import lax
from jax.experimental import pallas as pl
from jax.experimental.pallas import tpu as pltpu
from jax.experimental.pallas import tpu_sc as plsc

N = 10000
D = 128
E = 320000
NP = 10240
EP = 327680
EDGE_B = 128
N_TILES = 32
E_PER_TILE = EP // N_TILES
N_BATCH = E_PER_TILE // EDGE_B
ROWS_PER_TILE = NP // 16


NBUF = 2
NB0 = 158
NB1 = (2 * N_BATCH) - NB0


def _sc_conv_body(x_hbm, src_hbm, dst_hbm, out_hbm,
                  sidx0, sidx1, didx0, didx1, rows0, rows1, zbuf,
                  agg_sh, g0, g1, si0, si1, di0, di1):
    c = lax.axis_index("c")
    s = lax.axis_index("s")
    rows = (rows0, rows1)
    sidx = (sidx0, sidx1)
    didx = (didx0, didx1)
    gsems = (g0, g1)
    sisems = (si0, si1)
    disems = (di0, di1)

    nb = jnp.where(c == 0, NB0, NB1)
    eb = jnp.where(c == 0, s * NB0, 16 * NB0 + s * NB1) * EDGE_B

    def fire_idx(b, g):
        base = pl.multiple_of(eb + g * EDGE_B, EDGE_B)
        pltpu.async_copy(src_hbm.at[pl.ds(base, EDGE_B)], sidx[b], sisems[b])
        pltpu.async_copy(dst_hbm.at[pl.ds(base, EDGE_B)], didx[b], disems[b])

    def wait_sidx(b):
        pltpu.make_async_copy(src_hbm.at[pl.ds(0, EDGE_B)], sidx[b],
                              sisems[b]).wait()

    def wait_didx(b):
        pltpu.make_async_copy(dst_hbm.at[pl.ds(0, EDGE_B)], didx[b],
                              disems[b]).wait()

    fire_idx(0, 0)
    fire_idx(1, 1)
    wait_sidx(0)
    pltpu.async_copy(x_hbm.at[sidx0], rows0, g0)

    zrow = 64

    def zloop(i, carry):
        for j in range(D // 16):
            zbuf[i, pl.ds(16 * j, 16)] = jnp.zeros((16,), jnp.float32)
        return carry

    lax.fori_loop(0, zrow, zloop, 0)
    row0 = s * ROWS_PER_TILE
    for t in range(ROWS_PER_TILE // zrow):
        pltpu.sync_copy(zbuf, agg_sh.at[pl.ds(row0 + t * zrow, zrow)])
    plsc.subcore_barrier()

    def outer(o, carry):
        for b in range(NBUF):
            g = o * NBUF + b
            other = 1 - b
            pltpu.make_async_copy(x_hbm.at[sidx[b]], rows[b],
                                  gsems[b]).wait()

            @pl.when(g < nb - 1)
            def _():
                wait_sidx(other)
                pltpu.async_copy(x_hbm.at[sidx[other]], rows[other],
                                 gsems[other])

            wait_didx(b)
            pltpu.sync_copy(rows[b], agg_sh.at[didx[b]], add=True)

            @pl.when(g < nb - 2)
            def _():
                fire_idx(b, g + 2)
        return carry

    lax.fori_loop(0, nb // NBUF, outer, 0)
    plsc.subcore_barrier()
    out0 = c * NP + row0
    pltpu.sync_copy(agg_sh.at[pl.ds(row0, ROWS_PER_TILE)],
                    out_hbm.at[pl.ds(out0, ROWS_PER_TILE)])


@jax.jit
def _sc_conv(x_p, src_p, dst_p):
    mesh = plsc.VectorSubcoreMesh(core_axis_name="c", subcore_axis_name="s")
    return pl.kernel(
        _sc_conv_body,
        out_type=jax.ShapeDtypeStruct((2 * NP, D), jnp.float32),
        mesh=mesh,
        scratch_types=[
            pltpu.VMEM((EDGE_B,), jnp.int32),
            pltpu.VMEM((EDGE_B,), jnp.int32),
            pltpu.VMEM((EDGE_B,), jnp.int32),
            pltpu.VMEM((EDGE_B,), jnp.int32),
            pltpu.VMEM((EDGE_B, D), jnp.float32),
            pltpu.VMEM((EDGE_B, D), jnp.float32),
            pltpu.VMEM((64, D), jnp.float32),
            pltpu.VMEM_SHARED((NP, D), jnp.float32),
            pltpu.SemaphoreType.DMA,
            pltpu.SemaphoreType.DMA,
            pltpu.SemaphoreType.DMA,
            pltpu.SemaphoreType.DMA,
            pltpu.SemaphoreType.DMA,
            pltpu.SemaphoreType.DMA,
        ],
    )(x_p, src_p, dst_p)


def _tc_dense_body(agg_ref, x_ref, wrel_ref, brel_ref, wroot_ref, pwc_ref,
                   xn_ref, v_ref):
    agg = agg_ref[0:NP, :] + agg_ref[NP:2 * NP, :]
    xn = jnp.maximum(
        jnp.dot(agg, wrel_ref[...], preferred_element_type=jnp.float32)
        + brel_ref[...]
        + jnp.dot(x_ref[...], wroot_ref[...], preferred_element_type=jnp.float32),
        0.0)
    pwc = pwc_ref[...]
    nrm = jnp.sqrt(jnp.sum(pwc * pwc))
    s_col = jnp.dot(xn, pwc, preferred_element_type=jnp.float32)
    xn_ref[...] = xn
    v_ref[...] = jnp.tanh(s_col / nrm)


@jax.jit
def _tc_dense(agg, x, wrel, brel, wroot, pwc):
    f32 = jnp.float32
    return pl.pallas_call(
        _tc_dense_body,
        out_shape=[
            jax.ShapeDtypeStruct((NP, D), f32),
            jax.ShapeDtypeStruct((NP, 1), f32),
        ],
    )(agg, x, wrel, brel, wroot, pwc)


def _ukey(v):
    bits = lax.bitcast_convert_type(v, jnp.int32)
    return lax.bitcast_convert_type(
        bits ^ ((bits >> 31) | jnp.int32(-2147483648)), jnp.uint32)


def _search_value_theta(tie, ukey, r):
    def step(j, th):
        cand = th | (jnp.uint32(0x80000000) >> j.astype(jnp.uint32))
        cnt = jnp.sum(jnp.where(tie & (ukey >= cand), 1, 0))
        return jnp.where(cnt >= r, cand, th)
    return lax.fori_loop(0, 32, step, jnp.uint32(0))


def _search_index_tmax(tie, idx, r):
    def step(j, th):
        cand = th | (jnp.int32(1 << 14) >> j)
        cnt = jnp.sum(jnp.where(tie & (idx <= cand), 1, 0))
        return jnp.where(cnt <= r, cand, th)
    return lax.fori_loop(0, 15, step, jnp.int32(0))


def _make_select_body(nhist, k):
    def body(*refs):
        (xn_ref, vr_ref, vc_ref, ar_ref, ac_ref) = refs[:5]
        vh_r = [refs[5 + j] for j in range(nhist)]
        vh_c = [refs[5 + nhist + j] for j in range(nhist)]
        (xs_ref, sel_c_ref, pooled_ref) = refs[5 + 2 * nhist:]

        alive_r = ar_ref[...] > 0.0
        alive_c = ac_ref[...] > 0.0

        row_keys = [jnp.where(alive_r, _ukey(vr_ref[...]), jnp.uint32(0))]
        for vr in vh_r:
            row_keys.append(jnp.where(alive_r, _ukey(vr[...]), jnp.uint32(0)))
        idx_row = lax.broadcasted_iota(jnp.int32, (1, NP), 1)

        tie = alive_r
        r = jnp.int32(k)
        thetas = []
        for u in row_keys:
            th = _search_value_theta(tie, u, r)
            above = tie & (u > th)
            r = r - jnp.sum(jnp.where(above, 1, 0))
            tie = tie & (u == th)
            thetas.append(th)
        tmax = _search_index_tmax(tie, idx_row, r)

        col_keys = [jnp.where(alive_c, _ukey(vc_ref[...]), jnp.uint32(0))]
        for vc in vh_c:
            col_keys.append(jnp.where(alive_c, _ukey(vc[...]), jnp.uint32(0)))
        idx_col = lax.broadcasted_iota(jnp.int32, (NP, 1), 0)
        sel_c = idx_col <= tmax
        for u, th in zip(reversed(col_keys), reversed(thetas)):
            sel_c = (u > th) | ((u == th) & sel_c)
        sel_c = alive_c & sel_c

        xs = xn_ref[...] * jnp.where(sel_c, vc_ref[...], 0.0)
        xs_ref[...] = xs
        sel_c_ref[...] = jnp.where(sel_c, 1.0, 0.0)
        pmax = jnp.max(jnp.where(sel_c, xs, -jnp.inf), axis=0, keepdims=True)
        psum = jnp.sum(xs, axis=0, keepdims=True) * (1.0 / k)
        pooled_ref[...] = jnp.concatenate([pmax, psum], axis=1)

    return body


@functools.partial(jax.jit, static_argnames=("nhist", "k"))
def _tc_select(xn, v_r, v_c, alive_r, alive_c, vh_r, vh_c, *, nhist, k):
    f32 = jnp.float32
    out_shape = [
        jax.ShapeDtypeStruct((NP, D), f32),
        jax.ShapeDtypeStruct((NP, 1), f32),
        jax.ShapeDtypeStruct((1, 2 * D), f32),
    ]
    return pl.pallas_call(
        _make_select_body(nhist, k),
        out_shape=out_shape,
    )(xn, v_r, v_c, alive_r, alive_c, *vh_r, *vh_c)


def _head_body(p1, p2, p3, w1, b1, w2, b2, w3, b3,
               logits_ref, probs_ref, yhat_ref):
    xh = p1[...] + p2[...] + p3[...]
    h = jnp.maximum(jnp.dot(xh, w1[...], preferred_element_type=jnp.float32)
                    + b1[...], 0.0)
    h = jnp.maximum(jnp.dot(h, w2[...], preferred_element_type=jnp.float32)
                    + b2[...], 0.0)
    logits = jnp.dot(h, w3[...], preferred_element_type=jnp.float32) + b3[...]
    logits_ref[...] = logits
    mx = jnp.max(logits, axis=1, keepdims=True)
    ex = jnp.exp(logits - mx)
    probs_ref[...] = ex / jnp.sum(ex, axis=1, keepdims=True)
    iv = lax.broadcasted_iota(jnp.int32, (1, 2), 1)
    yhat_ref[...] = jnp.min(jnp.where(logits == mx, iv, jnp.int32(2)),
                            axis=1, keepdims=True)


@jax.jit
def _head(p1, p2, p3, w1, b1, w2, b2, w3, b3):
    f32 = jnp.float32
    return pl.pallas_call(
        _head_body,
        out_shape=[
            jax.ShapeDtypeStruct((1, 2), f32),
            jax.ShapeDtypeStruct((1, 2), f32),
            jax.ShapeDtypeStruct((1, 1), jnp.int32),
        ],
    )(p1, p2, p3, w1, b1, w2, b2, w3, b3)


def kernel(x, adj, Wrel1, brel1, Wroot1, pw1, Wrel2, brel2, Wroot2, pw2,
           Wrel3, brel3, Wroot3, pw3, W1, b1, W2, b2, W3, b3):
    f32 = jnp.float32
    x_p = jnp.zeros((NP, D), f32).at[:N].set(x)
    pad = jnp.full((EP - E,), NP - 1, jnp.int32)
    src_p = jnp.concatenate([adj[0].astype(jnp.int32), pad])
    dst_p = jnp.concatenate([adj[1].astype(jnp.int32), pad])
    alive_c = (jnp.arange(NP, dtype=jnp.int32)[:, None] < N).astype(f32)

    layers = [
        (Wrel1, brel1, Wroot1, pw1, 8000),
        (Wrel2, brel2, Wroot2, pw2, 6400),
        (Wrel3, brel3, Wroot3, pw3, 5120),
    ]
    vh_c = []
    pooled = []
    for i, (wrel, brel, wroot, pw, k) in enumerate(layers):
        agg = _sc_conv(x_p, src_p, dst_p)
        xn, v_c = _tc_dense(agg, x_p, wrel, brel[None, :], wroot, pw[:, None])
        v_r = jnp.reshape(v_c, (1, NP))
        alive_r = jnp.reshape(alive_c, (1, NP))
        vh_r = tuple(jnp.reshape(v, (1, NP)) for v in vh_c)
        x_p, alive_c, pld = _tc_select(xn, v_r, v_c, alive_r, alive_c,
                                       vh_r, tuple(vh_c), nhist=i, k=k)
        vh_c.insert(0, v_c)
        pooled.append(pld)

    logits, probs, yhat = _head(pooled[0], pooled[1], pooled[2],
                                W1, b1[None, :], W2, b2[None, :],
                                W3, b3[None, :])
    return (logits, probs, yhat)


# speedup vs baseline: 1.2346x; 1.2346x over previous
import functools
import math

import jax
import jax.numpy as jnp
from jax import lax
from jax.experimental import pallas as pl
from jax.experimental.pallas import tpu as pltpu
from jax.experimental.pallas import tpu_sc as plsc

N = 10000
D = 128
E = 320000
NP = 10240
EP = 327680
EDGE_B = 128
N_TILES = 32
E_PER_TILE = EP // N_TILES
N_BATCH = E_PER_TILE // EDGE_B
ROWS_PER_TILE = NP // 16


NBUF = 2
NB0 = 154
NB1 = (2 * N_BATCH) - NB0


def _sc_conv_body(x_hbm, src_hbm, dst_hbm, out_hbm,
                  sidx0, sidx1, didx0, didx1, rows0, rows1, zbuf,
                  agg_sh, g0, g1, si0, si1, di0, di1):
    c = lax.axis_index("c")
    s = lax.axis_index("s")
    rows = (rows0, rows1)
    sidx = (sidx0, sidx1)
    didx = (didx0, didx1)
    gsems = (g0, g1)
    sisems = (si0, si1)
    disems = (di0, di1)

    nb = jnp.where(c == 0, NB0, NB1)
    eb = jnp.where(c == 0, s * NB0, 16 * NB0 + s * NB1) * EDGE_B

    def fire_idx(b, g):
        base = pl.multiple_of(eb + g * EDGE_B, EDGE_B)
        pltpu.async_copy(src_hbm.at[pl.ds(base, EDGE_B)], sidx[b], sisems[b])
        pltpu.async_copy(dst_hbm.at[pl.ds(base, EDGE_B)], didx[b], disems[b])

    def wait_sidx(b):
        pltpu.make_async_copy(src_hbm.at[pl.ds(0, EDGE_B)], sidx[b],
                              sisems[b]).wait()

    def wait_didx(b):
        pltpu.make_async_copy(dst_hbm.at[pl.ds(0, EDGE_B)], didx[b],
                              disems[b]).wait()

    fire_idx(0, 0)
    fire_idx(1, 1)
    wait_sidx(0)
    pltpu.async_copy(x_hbm.at[sidx0], rows0, g0)

    zrow = 64

    def zloop(i, carry):
        for j in range(D // 16):
            zbuf[i, pl.ds(16 * j, 16)] = jnp.zeros((16,), jnp.float32)
        return carry

    lax.fori_loop(0, zrow, zloop, 0)
    row0 = s * ROWS_PER_TILE
    for t in range(ROWS_PER_TILE // zrow):
        pltpu.sync_copy(zbuf, agg_sh.at[pl.ds(row0 + t * zrow, zrow)])
    plsc.subcore_barrier()

    def outer(o, carry):
        for b in range(NBUF):
            g = o * NBUF + b
            other = 1 - b
            pltpu.make_async_copy(x_hbm.at[sidx[b]], rows[b],
                                  gsems[b]).wait()

            @pl.when(g < nb - 1)
            def _():
                wait_sidx(other)
                pltpu.async_copy(x_hbm.at[sidx[other]], rows[other],
                                 gsems[other])

            wait_didx(b)
            pltpu.sync_copy(rows[b], agg_sh.at[didx[b]], add=True)

            @pl.when(g < nb - 2)
            def _():
                fire_idx(b, g + 2)
        return carry

    lax.fori_loop(0, nb // NBUF, outer, 0)
    plsc.subcore_barrier()
    out0 = c * NP + row0
    pltpu.sync_copy(agg_sh.at[pl.ds(row0, ROWS_PER_TILE)],
                    out_hbm.at[pl.ds(out0, ROWS_PER_TILE)])


@jax.jit
def _sc_conv(x_p, src_p, dst_p):
    mesh = plsc.VectorSubcoreMesh(core_axis_name="c", subcore_axis_name="s")
    return pl.kernel(
        _sc_conv_body,
        out_type=jax.ShapeDtypeStruct((2 * NP, D), jnp.float32),
        mesh=mesh,
        scratch_types=[
            pltpu.VMEM((EDGE_B,), jnp.int32),
            pltpu.VMEM((EDGE_B,), jnp.int32),
            pltpu.VMEM((EDGE_B,), jnp.int32),
            pltpu.VMEM((EDGE_B,), jnp.int32),
            pltpu.VMEM((EDGE_B, D), jnp.float32),
            pltpu.VMEM((EDGE_B, D), jnp.float32),
            pltpu.VMEM((64, D), jnp.float32),
            pltpu.VMEM_SHARED((NP, D), jnp.float32),
            pltpu.SemaphoreType.DMA,
            pltpu.SemaphoreType.DMA,
            pltpu.SemaphoreType.DMA,
            pltpu.SemaphoreType.DMA,
            pltpu.SemaphoreType.DMA,
            pltpu.SemaphoreType.DMA,
        ],
    )(x_p, src_p, dst_p)


def _tc_dense_body(agg_ref, x_ref, wrel_ref, brel_ref, wroot_ref, pwc_ref,
                   xn_ref, v_ref):
    agg = agg_ref[0:NP, :] + agg_ref[NP:2 * NP, :]
    xn = jnp.maximum(
        jnp.dot(agg, wrel_ref[...], preferred_element_type=jnp.float32)
        + brel_ref[...]
        + jnp.dot(x_ref[...], wroot_ref[...], preferred_element_type=jnp.float32),
        0.0)
    pwc = pwc_ref[...]
    nrm = jnp.sqrt(jnp.sum(pwc * pwc))
    s_col = jnp.dot(xn, pwc, preferred_element_type=jnp.float32)
    xn_ref[...] = xn
    v_ref[...] = jnp.tanh(s_col / nrm)


@jax.jit
def _tc_dense(agg, x, wrel, brel, wroot, pwc):
    f32 = jnp.float32
    return pl.pallas_call(
        _tc_dense_body,
        out_shape=[
            jax.ShapeDtypeStruct((NP, D), f32),
            jax.ShapeDtypeStruct((NP, 1), f32),
        ],
    )(agg, x, wrel, brel, wroot, pwc)


def _ukey(v):
    bits = lax.bitcast_convert_type(v, jnp.int32)
    return lax.bitcast_convert_type(
        bits ^ ((bits >> 31) | jnp.int32(-2147483648)), jnp.uint32)


def _search_value_theta(tie, ukey, r):
    def step(j, th):
        cand = th | (jnp.uint32(0x80000000) >> j.astype(jnp.uint32))
        cnt = jnp.sum(jnp.where(tie & (ukey >= cand), 1, 0))
        return jnp.where(cnt >= r, cand, th)
    return lax.fori_loop(0, 32, step, jnp.uint32(0))


def _search_index_tmax(tie, idx, r):
    def step(j, th):
        cand = th | (jnp.int32(1 << 14) >> j)
        cnt = jnp.sum(jnp.where(tie & (idx <= cand), 1, 0))
        return jnp.where(cnt <= r, cand, th)
    return lax.fori_loop(0, 15, step, jnp.int32(0))


def _make_select_body(nhist, k):
    def body(*refs):
        (xn_ref, vr_ref, vc_ref, ar_ref, ac_ref) = refs[:5]
        vh_r = [refs[5 + j] for j in range(nhist)]
        vh_c = [refs[5 + nhist + j] for j in range(nhist)]
        (xs_ref, sel_c_ref, pooled_ref) = refs[5 + 2 * nhist:]

        alive_r = ar_ref[...] > 0.0
        alive_c = ac_ref[...] > 0.0

        row_keys = [jnp.where(alive_r, _ukey(vr_ref[...]), jnp.uint32(0))]
        for vr in vh_r:
            row_keys.append(jnp.where(alive_r, _ukey(vr[...]), jnp.uint32(0)))
        idx_row = lax.broadcasted_iota(jnp.int32, (1, NP), 1)

        tie = alive_r
        r = jnp.int32(k)
        thetas = []
        for u in row_keys:
            th = _search_value_theta(tie, u, r)
            above = tie & (u > th)
            r = r - jnp.sum(jnp.where(above, 1, 0))
            tie = tie & (u == th)
            thetas.append(th)
        tmax = _search_index_tmax(tie, idx_row, r)

        col_keys = [jnp.where(alive_c, _ukey(vc_ref[...]), jnp.uint32(0))]
        for vc in vh_c:
            col_keys.append(jnp.where(alive_c, _ukey(vc[...]), jnp.uint32(0)))
        idx_col = lax.broadcasted_iota(jnp.int32, (NP, 1), 0)
        sel_c = idx_col <= tmax
        for u, th in zip(reversed(col_keys), reversed(thetas)):
            sel_c = (u > th) | ((u == th) & sel_c)
        sel_c = alive_c & sel_c

        xs = xn_ref[...] * jnp.where(sel_c, vc_ref[...], 0.0)
        xs_ref[...] = xs
        sel_c_ref[...] = jnp.where(sel_c, 1.0, 0.0)
        pmax = jnp.max(jnp.where(sel_c, xs, -jnp.inf), axis=0, keepdims=True)
        psum = jnp.sum(xs, axis=0, keepdims=True) * (1.0 / k)
        pooled_ref[...] = jnp.concatenate([pmax, psum], axis=1)

    return body


@functools.partial(jax.jit, static_argnames=("nhist", "k"))
def _tc_select(xn, v_r, v_c, alive_r, alive_c, vh_r, vh_c, *, nhist, k):
    f32 = jnp.float32
    out_shape = [
        jax.ShapeDtypeStruct((NP, D), f32),
        jax.ShapeDtypeStruct((NP, 1), f32),
        jax.ShapeDtypeStruct((1, 2 * D), f32),
    ]
    return pl.pallas_call(
        _make_select_body(nhist, k),
        out_shape=out_shape,
    )(xn, v_r, v_c, alive_r, alive_c, *vh_r, *vh_c)


def _head_body(p1, p2, p3, w1, b1, w2, b2, w3, b3,
               logits_ref, probs_ref, yhat_ref):
    xh = p1[...] + p2[...] + p3[...]
    h = jnp.maximum(jnp.dot(xh, w1[...], preferred_element_type=jnp.float32)
                    + b1[...], 0.0)
    h = jnp.maximum(jnp.dot(h, w2[...], preferred_element_type=jnp.float32)
                    + b2[...], 0.0)
    logits = jnp.dot(h, w3[...], preferred_element_type=jnp.float32) + b3[...]
    logits_ref[...] = logits
    mx = jnp.max(logits, axis=1, keepdims=True)
    ex = jnp.exp(logits - mx)
    probs_ref[...] = ex / jnp.sum(ex, axis=1, keepdims=True)
    iv = lax.broadcasted_iota(jnp.int32, (1, 2), 1)
    yhat_ref[...] = jnp.min(jnp.where(logits == mx, iv, jnp.int32(2)),
                            axis=1, keepdims=True)


@jax.jit
def _head(p1, p2, p3, w1, b1, w2, b2, w3, b3):
    f32 = jnp.float32
    return pl.pallas_call(
        _head_body,
        out_shape=[
            jax.ShapeDtypeStruct((1, 2), f32),
            jax.ShapeDtypeStruct((1, 2), f32),
            jax.ShapeDtypeStruct((1, 1), jnp.int32),
        ],
    )(p1, p2, p3, w1, b1, w2, b2, w3, b3)


def kernel(x, adj, Wrel1, brel1, Wroot1, pw1, Wrel2, brel2, Wroot2, pw2,
           Wrel3, brel3, Wroot3, pw3, W1, b1, W2, b2, W3, b3):
    f32 = jnp.float32
    x_p = jnp.zeros((NP, D), f32).at[:N].set(x)
    pad = jnp.full((EP - E,), NP - 1, jnp.int32)
    src_p = jnp.concatenate([adj[0].astype(jnp.int32), pad])
    dst_p = jnp.concatenate([adj[1].astype(jnp.int32), pad])
    alive_c = (jnp.arange(NP, dtype=jnp.int32)[:, None] < N).astype(f32)

    layers = [
        (Wrel1, brel1, Wroot1, pw1, 8000),
        (Wrel2, brel2, Wroot2, pw2, 6400),
        (Wrel3, brel3, Wroot3, pw3, 5120),
    ]
    vh_c = []
    pooled = []
    for i, (wrel, brel, wroot, pw, k) in enumerate(layers):
        agg = _sc_conv(x_p, src_p, dst_p)
        xn, v_c = _tc_dense(agg, x_p, wrel, brel[None, :], wroot, pw[:, None])
        v_r = jnp.reshape(v_c, (1, NP))
        alive_r = jnp.reshape(alive_c, (1, NP))
        vh_r = tuple(jnp.reshape(v, (1, NP)) for v in vh_c)
        x_p, alive_c, pld = _tc_select(xn, v_r, v_c, alive_r, alive_c,
                                       vh_r, tuple(vh_c), nhist=i, k=k)
        vh_c.insert(0, v_c)
        pooled.append(pld)

    logits, probs, yhat = _head(pooled[0], pooled[1], pooled[2],
                                W1, b1[None, :], W2, b2[None, :],
                                W3, b3[None, :])
    return (logits, probs, yhat)


# speedup vs baseline: 1.3145x; 1.0647x over previous
import functools
import math

import jax
import jax.numpy as jnp
from jax import lax
from jax.experimental import pallas as pl
from jax.experimental.pallas import tpu as pltpu
from jax.experimental.pallas import tpu_sc as plsc

N = 10000
D = 128
E = 320000
NP = 10240
EP = 327680
EDGE_B = 128
N_TILES = 32
E_PER_TILE = EP // N_TILES
N_BATCH = E_PER_TILE // EDGE_B
ROWS_PER_TILE = NP // 16


NBUF = 2
NB0 = 150
NB1 = (2 * N_BATCH) - NB0


def _sc_conv_body(x_hbm, src_hbm, dst_hbm, out_hbm,
                  sidx0, sidx1, didx0, didx1, rows0, rows1, zbuf,
                  agg_sh, g0, g1, si0, si1, di0, di1):
    c = lax.axis_index("c")
    s = lax.axis_index("s")
    rows = (rows0, rows1)
    sidx = (sidx0, sidx1)
    didx = (didx0, didx1)
    gsems = (g0, g1)
    sisems = (si0, si1)
    disems = (di0, di1)

    nb = jnp.where(c == 0, NB0, NB1)
    eb = jnp.where(c == 0, s * NB0, 16 * NB0 + s * NB1) * EDGE_B

    def fire_idx(b, g):
        base = pl.multiple_of(eb + g * EDGE_B, EDGE_B)
        pltpu.async_copy(src_hbm.at[pl.ds(base, EDGE_B)], sidx[b], sisems[b])
        pltpu.async_copy(dst_hbm.at[pl.ds(base, EDGE_B)], didx[b], disems[b])

    def wait_sidx(b):
        pltpu.make_async_copy(src_hbm.at[pl.ds(0, EDGE_B)], sidx[b],
                              sisems[b]).wait()

    def wait_didx(b):
        pltpu.make_async_copy(dst_hbm.at[pl.ds(0, EDGE_B)], didx[b],
                              disems[b]).wait()

    fire_idx(0, 0)
    fire_idx(1, 1)
    wait_sidx(0)
    pltpu.async_copy(x_hbm.at[sidx0], rows0, g0)

    zrow = 64

    def zloop(i, carry):
        for j in range(D // 16):
            zbuf[i, pl.ds(16 * j, 16)] = jnp.zeros((16,), jnp.float32)
        return carry

    lax.fori_loop(0, zrow, zloop, 0)
    row0 = s * ROWS_PER_TILE
    for t in range(ROWS_PER_TILE // zrow):
        pltpu.sync_copy(zbuf, agg_sh.at[pl.ds(row0 + t * zrow, zrow)])
    plsc.subcore_barrier()

    def outer(o, carry):
        for b in range(NBUF):
            g = o * NBUF + b
            other = 1 - b
            pltpu.make_async_copy(x_hbm.at[sidx[b]], rows[b],
                                  gsems[b]).wait()

            @pl.when(g < nb - 1)
            def _():
                wait_sidx(other)
                pltpu.async_copy(x_hbm.at[sidx[other]], rows[other],
                                 gsems[other])

            wait_didx(b)
            pltpu.sync_copy(rows[b], agg_sh.at[didx[b]], add=True)

            @pl.when(g < nb - 2)
            def _():
                fire_idx(b, g + 2)
        return carry

    lax.fori_loop(0, nb // NBUF, outer, 0)
    plsc.subcore_barrier()
    out0 = c * NP + row0
    pltpu.sync_copy(agg_sh.at[pl.ds(row0, ROWS_PER_TILE)],
                    out_hbm.at[pl.ds(out0, ROWS_PER_TILE)])


@jax.jit
def _sc_conv(x_p, src_p, dst_p):
    mesh = plsc.VectorSubcoreMesh(core_axis_name="c", subcore_axis_name="s")
    return pl.kernel(
        _sc_conv_body,
        out_type=jax.ShapeDtypeStruct((2 * NP, D), jnp.float32),
        mesh=mesh,
        scratch_types=[
            pltpu.VMEM((EDGE_B,), jnp.int32),
            pltpu.VMEM((EDGE_B,), jnp.int32),
            pltpu.VMEM((EDGE_B,), jnp.int32),
            pltpu.VMEM((EDGE_B,), jnp.int32),
            pltpu.VMEM((EDGE_B, D), jnp.float32),
            pltpu.VMEM((EDGE_B, D), jnp.float32),
            pltpu.VMEM((64, D), jnp.float32),
            pltpu.VMEM_SHARED((NP, D), jnp.float32),
            pltpu.SemaphoreType.DMA,
            pltpu.SemaphoreType.DMA,
            pltpu.SemaphoreType.DMA,
            pltpu.SemaphoreType.DMA,
            pltpu.SemaphoreType.DMA,
            pltpu.SemaphoreType.DMA,
        ],
    )(x_p, src_p, dst_p)


def _tc_dense_body(agg_ref, x_ref, wrel_ref, brel_ref, wroot_ref, pwc_ref,
                   xn_ref, v_ref):
    agg = agg_ref[0:NP, :] + agg_ref[NP:2 * NP, :]
    xn = jnp.maximum(
        jnp.dot(agg, wrel_ref[...], preferred_element_type=jnp.float32)
        + brel_ref[...]
        + jnp.dot(x_ref[...], wroot_ref[...], preferred_element_type=jnp.float32),
        0.0)
    pwc = pwc_ref[...]
    nrm = jnp.sqrt(jnp.sum(pwc * pwc))
    s_col = jnp.dot(xn, pwc, preferred_element_type=jnp.float32)
    xn_ref[...] = xn
    v_ref[...] = jnp.tanh(s_col / nrm)


@jax.jit
def _tc_dense(agg, x, wrel, brel, wroot, pwc):
    f32 = jnp.float32
    return pl.pallas_call(
        _tc_dense_body,
        out_shape=[
            jax.ShapeDtypeStruct((NP, D), f32),
            jax.ShapeDtypeStruct((NP, 1), f32),
        ],
    )(agg, x, wrel, brel, wroot, pwc)


def _ukey(v):
    bits = lax.bitcast_convert_type(v, jnp.int32)
    return lax.bitcast_convert_type(
        bits ^ ((bits >> 31) | jnp.int32(-2147483648)), jnp.uint32)


def _search_value_theta(tie, ukey, r):
    def step(j, th):
        cand = th | (jnp.uint32(0x80000000) >> j.astype(jnp.uint32))
        cnt = jnp.sum(jnp.where(tie & (ukey >= cand), 1, 0))
        return jnp.where(cnt >= r, cand, th)
    return lax.fori_loop(0, 32, step, jnp.uint32(0))


def _search_index_tmax(tie, idx, r):
    def step(j, th):
        cand = th | (jnp.int32(1 << 14) >> j)
        cnt = jnp.sum(jnp.where(tie & (idx <= cand), 1, 0))
        return jnp.where(cnt <= r, cand, th)
    return lax.fori_loop(0, 15, step, jnp.int32(0))


def _make_select_body(nhist, k):
    def body(*refs):
        (xn_ref, vr_ref, vc_ref, ar_ref, ac_ref) = refs[:5]
        vh_r = [refs[5 + j] for j in range(nhist)]
        vh_c = [refs[5 + nhist + j] for j in range(nhist)]
        (xs_ref, sel_c_ref, pooled_ref) = refs[5 + 2 * nhist:]

        alive_r = ar_ref[...] > 0.0
        alive_c = ac_ref[...] > 0.0

        row_keys = [jnp.where(alive_r, _ukey(vr_ref[...]), jnp.uint32(0))]
        for vr in vh_r:
            row_keys.append(jnp.where(alive_r, _ukey(vr[...]), jnp.uint32(0)))
        idx_row = lax.broadcasted_iota(jnp.int32, (1, NP), 1)

        tie = alive_r
        r = jnp.int32(k)
        thetas = []
        for u in row_keys:
            th = _search_value_theta(tie, u, r)
            above = tie & (u > th)
            r = r - jnp.sum(jnp.where(above, 1, 0))
            tie = tie & (u == th)
            thetas.append(th)
        tmax = _search_index_tmax(tie, idx_row, r)

        col_keys = [jnp.where(alive_c, _ukey(vc_ref[...]), jnp.uint32(0))]
        for vc in vh_c:
            col_keys.append(jnp.where(alive_c, _ukey(vc[...]), jnp.uint32(0)))
        idx_col = lax.broadcasted_iota(jnp.int32, (NP, 1), 0)
        sel_c = idx_col <= tmax
        for u, th in zip(reversed(col_keys), reversed(thetas)):
            sel_c = (u > th) | ((u == th) & sel_c)
        sel_c = alive_c & sel_c

        xs = xn_ref[...] * jnp.where(sel_c, vc_ref[...], 0.0)
        xs_ref[...] = xs
        sel_c_ref[...] = jnp.where(sel_c, 1.0, 0.0)
        pmax = jnp.max(jnp.where(sel_c, xs, -jnp.inf), axis=0, keepdims=True)
        psum = jnp.sum(xs, axis=0, keepdims=True) * (1.0 / k)
        pooled_ref[...] = jnp.concatenate([pmax, psum], axis=1)

    return body


@functools.partial(jax.jit, static_argnames=("nhist", "k"))
def _tc_select(xn, v_r, v_c, alive_r, alive_c, vh_r, vh_c, *, nhist, k):
    f32 = jnp.float32
    out_shape = [
        jax.ShapeDtypeStruct((NP, D), f32),
        jax.ShapeDtypeStruct((NP, 1), f32),
        jax.ShapeDtypeStruct((1, 2 * D), f32),
    ]
    return pl.pallas_call(
        _make_select_body(nhist, k),
        out_shape=out_shape,
    )(xn, v_r, v_c, alive_r, alive_c, *vh_r, *vh_c)


def _head_body(p1, p2, p3, w1, b1, w2, b2, w3, b3,
               logits_ref, probs_ref, yhat_ref):
    xh = p1[...] + p2[...] + p3[...]
    h = jnp.maximum(jnp.dot(xh, w1[...], preferred_element_type=jnp.float32)
                    + b1[...], 0.0)
    h = jnp.maximum(jnp.dot(h, w2[...], preferred_element_type=jnp.float32)
                    + b2[...], 0.0)
    logits = jnp.dot(h, w3[...], preferred_element_type=jnp.float32) + b3[...]
    logits_ref[...] = logits
    mx = jnp.max(logits, axis=1, keepdims=True)
    ex = jnp.exp(logits - mx)
    probs_ref[...] = ex / jnp.sum(ex, axis=1, keepdims=True)
    iv = lax.broadcasted_iota(jnp.int32, (1, 2), 1)
    yhat_ref[...] = jnp.min(jnp.where(logits == mx, iv, jnp.int32(2)),
                            axis=1, keepdims=True)


@jax.jit
def _head(p1, p2, p3, w1, b1, w2, b2, w3, b3):
    f32 = jnp.float32
    return pl.pallas_call(
        _head_body,
        out_shape=[
            jax.ShapeDtypeStruct((1, 2), f32),
            jax.ShapeDtypeStruct((1, 2), f32),
            jax.ShapeDtypeStruct((1, 1), jnp.int32),
        ],
    )(p1, p2, p3, w1, b1, w2, b2, w3, b3)


def kernel(x, adj, Wrel1, brel1, Wroot1, pw1, Wrel2, brel2, Wroot2, pw2,
           Wrel3, brel3, Wroot3, pw3, W1, b1, W2, b2, W3, b3):
    f32 = jnp.float32
    x_p = jnp.zeros((NP, D), f32).at[:N].set(x)
    pad = jnp.full((EP - E,), NP - 1, jnp.int32)
    src_p = jnp.concatenate([adj[0].astype(jnp.int32), pad])
    dst_p = jnp.concatenate([adj[1].astype(jnp.int32), pad])
    alive_c = (jnp.arange(NP, dtype=jnp.int32)[:, None] < N).astype(f32)

    layers = [
        (Wrel1, brel1, Wroot1, pw1, 8000),
        (Wrel2, brel2, Wroot2, pw2, 6400),
        (Wrel3, brel3, Wroot3, pw3, 5120),
    ]
    vh_c = []
    pooled = []
    for i, (wrel, brel, wroot, pw, k) in enumerate(layers):
        agg = _sc_conv(x_p, src_p, dst_p)
        xn, v_c = _tc_dense(agg, x_p, wrel, brel[None, :], wroot, pw[:, None])
        v_r = jnp.reshape(v_c, (1, NP))
        alive_r = jnp.reshape(alive_c, (1, NP))
        vh_r = tuple(jnp.reshape(v, (1, NP)) for v in vh_c)
        x_p, alive_c, pld = _tc_select(xn, v_r, v_c, alive_r, alive_c,
                                       vh_r, tuple(vh_c), nhist=i, k=k)
        vh_c.insert(0, v_c)
        pooled.append(pld)

    logits, probs, yhat = _head(pooled[0], pooled[1], pooled[2],
                                W1, b1[None, :], W2, b2[None, :],
                                W3, b3[None, :])
    return (logits, probs, yhat)


# speedup vs baseline: 1.3170x; 1.0019x over previous
import functools
import math

import jax
import jax.numpy as jnp
from jax import lax
from jax.experimental import pallas as pl
from jax.experimental.pallas import tpu as pltpu
from jax.experimental.pallas import tpu_sc as plsc

N = 10000
D = 128
E = 320000
NP = 10240
EP = 327680
EDGE_B = 128
N_TILES = 32
E_PER_TILE = EP // N_TILES
N_BATCH = E_PER_TILE // EDGE_B
ROWS_PER_TILE = NP // 16


NBUF = 2
NB0 = 146
NB1 = (2 * N_BATCH) - NB0


def _sc_conv_body(x_hbm, src_hbm, dst_hbm, out_hbm,
                  sidx0, sidx1, didx0, didx1, rows0, rows1, zbuf,
                  agg_sh, g0, g1, si0, si1, di0, di1):
    c = lax.axis_index("c")
    s = lax.axis_index("s")
    rows = (rows0, rows1)
    sidx = (sidx0, sidx1)
    didx = (didx0, didx1)
    gsems = (g0, g1)
    sisems = (si0, si1)
    disems = (di0, di1)

    nb = jnp.where(c == 0, NB0, NB1)
    eb = jnp.where(c == 0, s * NB0, 16 * NB0 + s * NB1) * EDGE_B

    def fire_idx(b, g):
        base = pl.multiple_of(eb + g * EDGE_B, EDGE_B)
        pltpu.async_copy(src_hbm.at[pl.ds(base, EDGE_B)], sidx[b], sisems[b])
        pltpu.async_copy(dst_hbm.at[pl.ds(base, EDGE_B)], didx[b], disems[b])

    def wait_sidx(b):
        pltpu.make_async_copy(src_hbm.at[pl.ds(0, EDGE_B)], sidx[b],
                              sisems[b]).wait()

    def wait_didx(b):
        pltpu.make_async_copy(dst_hbm.at[pl.ds(0, EDGE_B)], didx[b],
                              disems[b]).wait()

    fire_idx(0, 0)
    fire_idx(1, 1)
    wait_sidx(0)
    pltpu.async_copy(x_hbm.at[sidx0], rows0, g0)

    zrow = 64

    def zloop(i, carry):
        for j in range(D // 16):
            zbuf[i, pl.ds(16 * j, 16)] = jnp.zeros((16,), jnp.float32)
        return carry

    lax.fori_loop(0, zrow, zloop, 0)
    row0 = s * ROWS_PER_TILE
    for t in range(ROWS_PER_TILE // zrow):
        pltpu.sync_copy(zbuf, agg_sh.at[pl.ds(row0 + t * zrow, zrow)])
    plsc.subcore_barrier()

    def outer(o, carry):
        for b in range(NBUF):
            g = o * NBUF + b
            other = 1 - b
            pltpu.make_async_copy(x_hbm.at[sidx[b]], rows[b],
                                  gsems[b]).wait()

            @pl.when(g < nb - 1)
            def _():
                wait_sidx(other)
                pltpu.async_copy(x_hbm.at[sidx[other]], rows[other],
                                 gsems[other])

            wait_didx(b)
            pltpu.sync_copy(rows[b], agg_sh.at[didx[b]], add=True)

            @pl.when(g < nb - 2)
            def _():
                fire_idx(b, g + 2)
        return carry

    lax.fori_loop(0, nb // NBUF, outer, 0)
    plsc.subcore_barrier()
    out0 = c * NP + row0
    pltpu.sync_copy(agg_sh.at[pl.ds(row0, ROWS_PER_TILE)],
                    out_hbm.at[pl.ds(out0, ROWS_PER_TILE)])


@jax.jit
def _sc_conv(x_p, src_p, dst_p):
    mesh = plsc.VectorSubcoreMesh(core_axis_name="c", subcore_axis_name="s")
    return pl.kernel(
        _sc_conv_body,
        out_type=jax.ShapeDtypeStruct((2 * NP, D), jnp.float32),
        mesh=mesh,
        scratch_types=[
            pltpu.VMEM((EDGE_B,), jnp.int32),
            pltpu.VMEM((EDGE_B,), jnp.int32),
            pltpu.VMEM((EDGE_B,), jnp.int32),
            pltpu.VMEM((EDGE_B,), jnp.int32),
            pltpu.VMEM((EDGE_B, D), jnp.float32),
            pltpu.VMEM((EDGE_B, D), jnp.float32),
            pltpu.VMEM((64, D), jnp.float32),
            pltpu.VMEM_SHARED((NP, D), jnp.float32),
            pltpu.SemaphoreType.DMA,
            pltpu.SemaphoreType.DMA,
            pltpu.SemaphoreType.DMA,
            pltpu.SemaphoreType.DMA,
            pltpu.SemaphoreType.DMA,
            pltpu.SemaphoreType.DMA,
        ],
    )(x_p, src_p, dst_p)


def _tc_dense_body(agg_ref, x_ref, wrel_ref, brel_ref, wroot_ref, pwc_ref,
                   xn_ref, v_ref):
    agg = agg_ref[0:NP, :] + agg_ref[NP:2 * NP, :]
    xn = jnp.maximum(
        jnp.dot(agg, wrel_ref[...], preferred_element_type=jnp.float32)
        + brel_ref[...]
        + jnp.dot(x_ref[...], wroot_ref[...], preferred_element_type=jnp.float32),
        0.0)
    pwc = pwc_ref[...]
    nrm = jnp.sqrt(jnp.sum(pwc * pwc))
    s_col = jnp.dot(xn, pwc, preferred_element_type=jnp.float32)
    xn_ref[...] = xn
    v_ref[...] = jnp.tanh(s_col / nrm)


@jax.jit
def _tc_dense(agg, x, wrel, brel, wroot, pwc):
    f32 = jnp.float32
    return pl.pallas_call(
        _tc_dense_body,
        out_shape=[
            jax.ShapeDtypeStruct((NP, D), f32),
            jax.ShapeDtypeStruct((NP, 1), f32),
        ],
    )(agg, x, wrel, brel, wroot, pwc)


def _ukey(v):
    bits = lax.bitcast_convert_type(v, jnp.int32)
    return lax.bitcast_convert_type(
        bits ^ ((bits >> 31) | jnp.int32(-2147483648)), jnp.uint32)


def _search_value_theta(tie, ukey, r):
    def step(j, th):
        cand = th | (jnp.uint32(0x80000000) >> j.astype(jnp.uint32))
        cnt = jnp.sum(jnp.where(tie & (ukey >= cand), 1, 0))
        return jnp.where(cnt >= r, cand, th)
    return lax.fori_loop(0, 32, step, jnp.uint32(0))


def _search_index_tmax(tie, idx, r):
    def step(j, th):
        cand = th | (jnp.int32(1 << 14) >> j)
        cnt = jnp.sum(jnp.where(tie & (idx <= cand), 1, 0))
        return jnp.where(cnt <= r, cand, th)
    return lax.fori_loop(0, 15, step, jnp.int32(0))


def _make_select_body(nhist, k):
    def body(*refs):
        (xn_ref, vr_ref, vc_ref, ar_ref, ac_ref) = refs[:5]
        vh_r = [refs[5 + j] for j in range(nhist)]
        vh_c = [refs[5 + nhist + j] for j in range(nhist)]
        (xs_ref, sel_c_ref, pooled_ref) = refs[5 + 2 * nhist:]

        alive_r = ar_ref[...] > 0.0
        alive_c = ac_ref[...] > 0.0

        row_keys = [jnp.where(alive_r, _ukey(vr_ref[...]), jnp.uint32(0))]
        for vr in vh_r:
            row_keys.append(jnp.where(alive_r, _ukey(vr[...]), jnp.uint32(0)))
        idx_row = lax.broadcasted_iota(jnp.int32, (1, NP), 1)

        tie = alive_r
        r = jnp.int32(k)
        thetas = []
        for u in row_keys:
            th = _search_value_theta(tie, u, r)
            above = tie & (u > th)
            r = r - jnp.sum(jnp.where(above, 1, 0))
            tie = tie & (u == th)
            thetas.append(th)
        tmax = _search_index_tmax(tie, idx_row, r)

        col_keys = [jnp.where(alive_c, _ukey(vc_ref[...]), jnp.uint32(0))]
        for vc in vh_c:
            col_keys.append(jnp.where(alive_c, _ukey(vc[...]), jnp.uint32(0)))
        idx_col = lax.broadcasted_iota(jnp.int32, (NP, 1), 0)
        sel_c = idx_col <= tmax
        for u, th in zip(reversed(col_keys), reversed(thetas)):
            sel_c = (u > th) | ((u == th) & sel_c)
        sel_c = alive_c & sel_c

        xs = xn_ref[...] * jnp.where(sel_c, vc_ref[...], 0.0)
        xs_ref[...] = xs
        sel_c_ref[...] = jnp.where(sel_c, 1.0, 0.0)
        pmax = jnp.max(jnp.where(sel_c, xs, -jnp.inf), axis=0, keepdims=True)
        psum = jnp.sum(xs, axis=0, keepdims=True) * (1.0 / k)
        pooled_ref[...] = jnp.concatenate([pmax, psum], axis=1)

    return body


@functools.partial(jax.jit, static_argnames=("nhist", "k"))
def _tc_select(xn, v_r, v_c, alive_r, alive_c, vh_r, vh_c, *, nhist, k):
    f32 = jnp.float32
    out_shape = [
        jax.ShapeDtypeStruct((NP, D), f32),
        jax.ShapeDtypeStruct((NP, 1), f32),
        jax.ShapeDtypeStruct((1, 2 * D), f32),
    ]
    return pl.pallas_call(
        _make_select_body(nhist, k),
        out_shape=out_shape,
    )(xn, v_r, v_c, alive_r, alive_c, *vh_r, *vh_c)


def _head_body(p1, p2, p3, w1, b1, w2, b2, w3, b3,
               logits_ref, probs_ref, yhat_ref):
    xh = p1[...] + p2[...] + p3[...]
    h = jnp.maximum(jnp.dot(xh, w1[...], preferred_element_type=jnp.float32)
                    + b1[...], 0.0)
    h = jnp.maximum(jnp.dot(h, w2[...], preferred_element_type=jnp.float32)
                    + b2[...], 0.0)
    logits = jnp.dot(h, w3[...], preferred_element_type=jnp.float32) + b3[...]
    logits_ref[...] = logits
    mx = jnp.max(logits, axis=1, keepdims=True)
    ex = jnp.exp(logits - mx)
    probs_ref[...] = ex / jnp.sum(ex, axis=1, keepdims=True)
    iv = lax.broadcasted_iota(jnp.int32, (1, 2), 1)
    yhat_ref[...] = jnp.min(jnp.where(logits == mx, iv, jnp.int32(2)),
                            axis=1, keepdims=True)


@jax.jit
def _head(p1, p2, p3, w1, b1, w2, b2, w3, b3):
    f32 = jnp.float32
    return pl.pallas_call(
        _head_body,
        out_shape=[
            jax.ShapeDtypeStruct((1, 2), f32),
            jax.ShapeDtypeStruct((1, 2), f32),
            jax.ShapeDtypeStruct((1, 1), jnp.int32),
        ],
    )(p1, p2, p3, w1, b1, w2, b2, w3, b3)


def kernel(x, adj, Wrel1, brel1, Wroot1, pw1, Wrel2, brel2, Wroot2, pw2,
           Wrel3, brel3, Wroot3, pw3, W1, b1, W2, b2, W3, b3):
    f32 = jnp.float32
    x_p = jnp.zeros((NP, D), f32).at[:N].set(x)
    pad = jnp.full((EP - E,), NP - 1, jnp.int32)
    src_p = jnp.concatenate([adj[0].astype(jnp.int32), pad])
    dst_p = jnp.concatenate([adj[1].astype(jnp.int32), pad])
    alive_c = (jnp.arange(NP, dtype=jnp.int32)[:, None] < N).astype(f32)

    layers = [
        (Wrel1, brel1, Wroot1, pw1, 8000),
        (Wrel2, brel2, Wroot2, pw2, 6400),
        (Wrel3, brel3, Wroot3, pw3, 5120),
    ]
    vh_c = []
    pooled = []
    for i, (wrel, brel, wroot, pw, k) in enumerate(layers):
        agg = _sc_conv(x_p, src_p, dst_p)
        xn, v_c = _tc_dense(agg, x_p, wrel, brel[None, :], wroot, pw[:, None])
        v_r = jnp.reshape(v_c, (1, NP))
        alive_r = jnp.reshape(alive_c, (1, NP))
        vh_r = tuple(jnp.reshape(v, (1, NP)) for v in vh_c)
        x_p, alive_c, pld = _tc_select(xn, v_r, v_c, alive_r, alive_c,
                                       vh_r, tuple(vh_c), nhist=i, k=k)
        vh_c.insert(0, v_c)
        pooled.append(pld)

    logits, probs, yhat = _head(pooled[0], pooled[1], pooled[2],
                                W1, b1[None, :], W2, b2[None, :],
                                W3, b3[None, :])
    return (logits, probs, yhat)
